# baseline (speedup 1.0000x reference)
import jax
import jax.numpy as jnp
from jax.experimental import pallas as pl
from jax.experimental.pallas import tpu as pltpu

_N_PAPER, _N_AUTHOR, _N_FIELD = 100000, 50000, 10000
_EMBED = 128
_CHUNK = 25000
_SLOTS = 4


def _chunk_list():
    chunks = []
    for t, n in ((2, _N_FIELD), (1, _N_AUTHOR), (0, _N_PAPER)):
        off = 0
        while off < n:
            rows = min(_CHUNK, n - off)
            chunks.append((t, off, rows))
            off += rows
    return chunks


def _dma_pipeline(p_in, a_in, f_in, p_out, a_out, f_out, *scratch):
    bufs = scratch[:_SLOTS]
    sins = scratch[_SLOTS:2 * _SLOTS]
    souts = scratch[2 * _SLOTS:]
    srcs = (p_in, a_in, f_in)
    dsts = (p_out, a_out, f_out)
    chunks = _chunk_list()
    n = len(chunks)

    def in_copy(i):
        t, off, rows = chunks[i]
        return pltpu.make_async_copy(
            srcs[t].at[pl.ds(off, rows), :],
            bufs[i % _SLOTS].at[pl.ds(0, rows), :],
            sins[i % _SLOTS],
        )

    def out_copy(i):
        t, off, rows = chunks[i]
        return pltpu.make_async_copy(
            bufs[i % _SLOTS].at[pl.ds(0, rows), :],
            dsts[t].at[pl.ds(off, rows), :],
            souts[i % _SLOTS],
        )

    depth = _SLOTS // 2
    for i in range(min(depth, n)):
        in_copy(i).start()
    for i in range(n):
        j = i + depth
        if j < n:
            if j - _SLOTS >= 0:
                out_copy(j - _SLOTS).wait()
            in_copy(j).start()
        in_copy(i).wait()
        out_copy(i).start()
    for i in range(max(0, n - 2 * depth), n):
        out_copy(i).wait()


def kernel(embed_paper, embed_author, embed_field):
    return pl.pallas_call(
        _dma_pipeline,
        in_specs=[pl.BlockSpec(memory_space=pltpu.MemorySpace.HBM)] * 3,
        out_specs=(pl.BlockSpec(memory_space=pltpu.MemorySpace.HBM),) * 3,
        out_shape=tuple(
            jax.ShapeDtypeStruct(x.shape, x.dtype)
            for x in (embed_paper, embed_author, embed_field)
        ),
        scratch_shapes=(
            [pltpu.VMEM((_CHUNK, _EMBED), jnp.float32)] * _SLOTS
            + [pltpu.SemaphoreType.DMA] * (2 * _SLOTS)
        ),
    )(embed_paper, embed_author, embed_field)


# speedup vs baseline: 1.0159x; 1.0159x over previous
import jax
import jax.numpy as jnp
from jax.experimental import pallas as pl
from jax.experimental.pallas import tpu as pltpu

_N_PAPER, _N_AUTHOR, _N_FIELD = 100000, 50000, 10000
_EMBED = 128
_CHUNK = 25000
_SLOTS = 4


def _chunk_list():
    chunks = []
    for t, n in enumerate((_N_PAPER, _N_AUTHOR, _N_FIELD)):
        off = 0
        while off < n:
            rows = min(_CHUNK, n - off)
            chunks.append((t, off, rows))
            off += rows
    return chunks


def _dma_pipeline(p_in, a_in, f_in, p_out, a_out, f_out, *scratch):
    bufs = scratch[:_SLOTS]
    sins = scratch[_SLOTS:2 * _SLOTS]
    souts = scratch[2 * _SLOTS:]
    srcs = (p_in, a_in, f_in)
    dsts = (p_out, a_out, f_out)
    chunks = _chunk_list()
    n = len(chunks)

    def in_copy(i):
        t, off, rows = chunks[i]
        return pltpu.make_async_copy(
            srcs[t].at[pl.ds(off, rows), :],
            bufs[i % _SLOTS].at[pl.ds(0, rows), :],
            sins[i % _SLOTS],
        )

    def out_copy(i):
        t, off, rows = chunks[i]
        return pltpu.make_async_copy(
            bufs[i % _SLOTS].at[pl.ds(0, rows), :],
            dsts[t].at[pl.ds(off, rows), :],
            souts[i % _SLOTS],
        )

    depth = _SLOTS // 2
    for i in range(min(depth, n)):
        in_copy(i).start()
    for i in range(n):
        j = i + depth
        if j < n:
            if j - _SLOTS >= 0:
                out_copy(j - _SLOTS).wait()
            in_copy(j).start()
        in_copy(i).wait()
        out_copy(i).start()
    for i in range(max(0, n - 2 * depth), n):
        out_copy(i).wait()


def kernel(embed_paper, embed_author, embed_field):
    return pl.pallas_call(
        _dma_pipeline,
        in_specs=[pl.BlockSpec(memory_space=pltpu.MemorySpace.HBM)] * 3,
        out_specs=(pl.BlockSpec(memory_space=pltpu.MemorySpace.HBM),) * 3,
        out_shape=tuple(
            jax.ShapeDtypeStruct(x.shape, x.dtype)
            for x in (embed_paper, embed_author, embed_field)
        ),
        scratch_shapes=(
            [pltpu.VMEM((_CHUNK, _EMBED), jnp.float32)] * _SLOTS
            + [pltpu.SemaphoreType.DMA] * (2 * _SLOTS)
        ),
    )(embed_paper, embed_author, embed_field)
